# baseline (speedup 1.0000x reference)
import functools

import jax
import jax.numpy as jnp
from jax import lax
from jax.experimental import pallas as pl
from jax.experimental.pallas import tpu as pltpu
from jax.experimental.pallas import tpu_sc as plsc

NC, NS = 2, 16
NW = NC * NS
L = 16

_T = 100000
_KF = 6
_MF = 4
_J = _KF * _KF * _MF
_N_EDGES = 120000
_N_TRIS = 200000

G_CHUNK = 18752
G_LASTW = _T * _KF - (NW - 1) * G_CHUNK
S_CHUNK = 12504
S_LASTW = _T * _MF - (NW - 1) * S_CHUNK
ACC_PAD = 200064
ACC_TILE = ACC_PAD // NS
TAB_CH = 7504
TAB_LAST = _N_EDGES - (NS - 1) * TAB_CH


def _stage_table(hbm, bounce, spmem, off, n):
    pltpu.sync_copy(hbm.at[pl.ds(off, n)], bounce.at[pl.ds(0, n)])
    pltpu.sync_copy(bounce.at[pl.ds(0, n)], spmem.at[pl.ds(off, n)])


def _gather_one(tab, idx_hbm, out_hbm, idx_v, val_v, base, n):
    pltpu.sync_copy(idx_hbm.at[pl.ds(base, n)], idx_v.at[pl.ds(0, n)])
    pltpu.sync_copy(tab.at[idx_v.at[pl.ds(0, n)]], val_v.at[pl.ds(0, n)])
    pltpu.sync_copy(val_v.at[pl.ds(0, n)], out_hbm.at[pl.ds(base, n)])


def _gather_body(kc, lc, kidx, lidx, outk, outl, idx_v, val_v, tabk, tabl):
    c = lax.axis_index("c")
    s = lax.axis_index("s")
    wid = s * NC + c
    toff = s * TAB_CH

    @pl.when(s < NS - 1)
    def _():
        _stage_table(kc, val_v, tabk, toff, TAB_CH)
        _stage_table(lc, val_v, tabl, toff, TAB_CH)

    @pl.when(s == NS - 1)
    def _():
        _stage_table(kc, val_v, tabk, toff, TAB_LAST)
        _stage_table(lc, val_v, tabl, toff, TAB_LAST)

    plsc.subcore_barrier()
    base = wid * G_CHUNK

    @pl.when(wid < NW - 1)
    def _():
        _gather_one(tabk, kidx, outk, idx_v, val_v, base, G_CHUNK)
        _gather_one(tabl, lidx, outl, idx_v, val_v, base, G_CHUNK)

    @pl.when(wid == NW - 1)
    def _():
        _gather_one(tabk, kidx, outk, idx_v, val_v, base, G_LASTW)
        _gather_one(tabl, lidx, outl, idx_v, val_v, base, G_LASTW)


@functools.cache
def _gather():
    return pl.kernel(
        _gather_body,
        out_type=(jax.ShapeDtypeStruct((_T * _KF,), jnp.float32),
                  jax.ShapeDtypeStruct((_T * _KF,), jnp.float32)),
        mesh=plsc.VectorSubcoreMesh(core_axis_name="c", subcore_axis_name="s",
                                    num_cores=NC, num_subcores=NS),
        scratch_types=[pltpu.VMEM((G_CHUNK,), jnp.int32),
                       pltpu.VMEM((G_CHUNK,), jnp.float32),
                       pltpu.VMEM_SHARED((_N_EDGES,), jnp.float32),
                       pltpu.VMEM_SHARED((_N_EDGES,), jnp.float32)],
    )


_BT = 8192


def _contract_body(tp_ref, kat_ref, kpar_ref, lat_ref, lpar_ref, mpar_ref,
                   out_ref):
    f32 = jnp.float32
    a_sel = (lax.broadcasted_iota(jnp.int32, (_J, _KF), 0) // (_KF * _MF)
             == lax.broadcasted_iota(jnp.int32, (_J, _KF), 1)).astype(f32)
    b_sel = ((lax.broadcasted_iota(jnp.int32, (_J, _KF), 0) // _MF) % _KF
             == lax.broadcasted_iota(jnp.int32, (_J, _KF), 1)).astype(f32)
    e_sel = (lax.broadcasted_iota(jnp.int32, (_MF, _J), 1) % _MF
             == lax.broadcasted_iota(jnp.int32, (_MF, _J), 0)).astype(f32)

    kp = kat_ref[...] * kpar_ref[...]
    lp = lat_ref[...] * lpar_ref[...]
    k_ext = jnp.dot(a_sel, kp, preferred_element_type=f32)
    l_ext = jnp.dot(b_sel, lp, preferred_element_type=f32)
    prod = tp_ref[...] * k_ext * l_ext
    out_ref[...] = (jnp.dot(e_sel, prod, preferred_element_type=f32)
                    * mpar_ref[...])


def _contract(tp_t, kat_t, kpar_t, lat_t, lpar_t, mpar_t):
    grid = (_T + _BT - 1) // _BT
    return pl.pallas_call(
        _contract_body,
        grid=(grid,),
        in_specs=[
            pl.BlockSpec((_J, _BT), lambda i: (0, i)),
            pl.BlockSpec((_KF, _BT), lambda i: (0, i)),
            pl.BlockSpec((_KF, _BT), lambda i: (0, i)),
            pl.BlockSpec((_KF, _BT), lambda i: (0, i)),
            pl.BlockSpec((_KF, _BT), lambda i: (0, i)),
            pl.BlockSpec((_MF, _BT), lambda i: (0, i)),
        ],
        out_specs=pl.BlockSpec((_MF, _BT), lambda i: (0, i)),
        out_shape=jax.ShapeDtypeStruct((_MF, _T), jnp.float32),
        compiler_params=pltpu.CompilerParams(
            dimension_semantics=("arbitrary",)),
    )(tp_t, kat_t, kpar_t, lat_t, lpar_t, mpar_t)


def _scatter_body(vals, sidx, zeros, out, idx_v, val_v, acc):
    c = lax.axis_index("c")
    s = lax.axis_index("s")
    wid = s * NC + c
    base = wid * S_CHUNK
    pltpu.sync_copy(zeros.at[pl.ds(s * ACC_TILE, ACC_TILE)],
                    val_v.at[pl.ds(0, ACC_TILE)])
    pltpu.sync_copy(val_v.at[pl.ds(0, ACC_TILE)],
                    acc.at[pl.ds(s * ACC_TILE, ACC_TILE)])

    @pl.when(wid < NW - 1)
    def _():
        pltpu.sync_copy(sidx.at[pl.ds(base, S_CHUNK)], idx_v)
        pltpu.sync_copy(vals.at[pl.ds(base, S_CHUNK)], val_v)

    @pl.when(wid == NW - 1)
    def _():
        pltpu.sync_copy(sidx.at[pl.ds(base, S_LASTW)],
                        idx_v.at[pl.ds(0, S_LASTW)])
        pltpu.sync_copy(vals.at[pl.ds(base, S_LASTW)],
                        val_v.at[pl.ds(0, S_LASTW)])
        for i in range((S_CHUNK - S_LASTW) // L):
            idx_v[pl.ds(S_LASTW + i * L, L)] = jnp.zeros((L,), jnp.int32)
            val_v[pl.ds(S_LASTW + i * L, L)] = jnp.zeros((L,), jnp.float32)

    plsc.subcore_barrier()
    pltpu.sync_copy(val_v, acc.at[idx_v], add=True)
    plsc.subcore_barrier()
    pltpu.sync_copy(acc.at[pl.ds(s * ACC_TILE, ACC_TILE)],
                    val_v.at[pl.ds(0, ACC_TILE)])
    pltpu.sync_copy(val_v.at[pl.ds(0, ACC_TILE)],
                    out.at[pl.ds(c * ACC_PAD + s * ACC_TILE, ACC_TILE)])


@functools.cache
def _scatter():
    return pl.kernel(
        _scatter_body,
        out_type=jax.ShapeDtypeStruct((NC * ACC_PAD,), jnp.float32),
        mesh=plsc.VectorSubcoreMesh(core_axis_name="c", subcore_axis_name="s",
                                    num_cores=NC, num_subcores=NS),
        scratch_types=[pltpu.VMEM((S_CHUNK,), jnp.int32),
                       pltpu.VMEM((S_CHUNK,), jnp.float32),
                       pltpu.VMEM_SHARED((ACC_PAD,), jnp.float32)],
    )


def _sum_body(p_ref, o_ref):
    o_ref[...] = p_ref[0, :] + p_ref[1, :]


def _sum_partials(partials):
    return pl.pallas_call(
        _sum_body,
        in_specs=[pl.BlockSpec((NC, ACC_PAD), lambda: (0, 0))],
        out_specs=pl.BlockSpec((ACC_PAD,), lambda: (0,)),
        out_shape=jax.ShapeDtypeStruct((ACC_PAD,), jnp.float32),
    )(partials)


def kernel(k_cochain, l_cochain, k_face_idx, k_face_parity, l_face_idx,
           l_face_parity, m_face_idx, m_face_parity, triple_prod):
    kidx = k_face_idx.T.reshape(-1).astype(jnp.int32)
    lidx = l_face_idx.T.reshape(-1).astype(jnp.int32)
    gk, gl = _gather()(k_cochain, l_cochain, kidx, lidx)
    kat_t = gk.reshape(_KF, _T)
    lat_t = gl.reshape(_KF, _T)

    tp_t = jnp.transpose(triple_prod, (1, 2, 3, 0)).reshape(_J, _T)
    mv_t = _contract(tp_t, kat_t, k_face_parity.T, lat_t, l_face_parity.T,
                     m_face_parity.T)

    vals = mv_t.reshape(-1)
    sidx = m_face_idx.T.reshape(-1).astype(jnp.int32)
    zeros = jnp.zeros((ACC_PAD,), jnp.float32)
    partials = _scatter()(vals, sidx, zeros).reshape(NC, ACC_PAD)
    return _sum_partials(partials)[:_N_TRIS]


# speedup vs baseline: 1.0147x; 1.0147x over previous
import functools

import jax
import jax.numpy as jnp
from jax import lax
from jax.experimental import pallas as pl
from jax.experimental.pallas import tpu as pltpu
from jax.experimental.pallas import tpu_sc as plsc

NC, NS = 2, 16
NW = NC * NS
L = 16

_T = 100000
_KF = 6
_MF = 4
_J = _KF * _KF * _MF
_N_EDGES = 120000
_N_TRIS = 200000

G_CHUNK = 18752
G_LASTW = _T * _KF - (NW - 1) * G_CHUNK
S_CHUNK = 12504
S_LASTW = _T * _MF - (NW - 1) * S_CHUNK
ACC_PAD = 200064
ACC_TILE = ACC_PAD // NS
TAB_CH = 7504
TAB_LAST = _N_EDGES - (NS - 1) * TAB_CH


def _stage_table(hbm, bounce, spmem, off, n):
    pltpu.sync_copy(hbm.at[pl.ds(off, n)], bounce.at[pl.ds(0, n)])
    pltpu.sync_copy(bounce.at[pl.ds(0, n)], spmem.at[pl.ds(off, n)])


def _gather_one(tab, idx_hbm, out_hbm, idx_v, val_v, base, n):
    pltpu.sync_copy(idx_hbm.at[pl.ds(base, n)], idx_v.at[pl.ds(0, n)])
    pltpu.sync_copy(tab.at[idx_v.at[pl.ds(0, n)]], val_v.at[pl.ds(0, n)])
    pltpu.sync_copy(val_v.at[pl.ds(0, n)], out_hbm.at[pl.ds(base, n)])


def _gather_body(kc, lc, kidx, lidx, outk, outl, idx_v, val_v, tabk, tabl):
    c = lax.axis_index("c")
    s = lax.axis_index("s")
    wid = s * NC + c
    toff = s * TAB_CH

    @pl.when(s < NS - 1)
    def _():
        _stage_table(kc, val_v, tabk, toff, TAB_CH)
        _stage_table(lc, val_v, tabl, toff, TAB_CH)

    @pl.when(s == NS - 1)
    def _():
        _stage_table(kc, val_v, tabk, toff, TAB_LAST)
        _stage_table(lc, val_v, tabl, toff, TAB_LAST)

    plsc.subcore_barrier()
    base = wid * G_CHUNK

    @pl.when(wid < NW - 1)
    def _():
        _gather_one(tabk, kidx, outk, idx_v, val_v, base, G_CHUNK)
        _gather_one(tabl, lidx, outl, idx_v, val_v, base, G_CHUNK)

    @pl.when(wid == NW - 1)
    def _():
        _gather_one(tabk, kidx, outk, idx_v, val_v, base, G_LASTW)
        _gather_one(tabl, lidx, outl, idx_v, val_v, base, G_LASTW)


@functools.cache
def _gather():
    return pl.kernel(
        _gather_body,
        out_type=(jax.ShapeDtypeStruct((_T * _KF,), jnp.float32),
                  jax.ShapeDtypeStruct((_T * _KF,), jnp.float32)),
        mesh=plsc.VectorSubcoreMesh(core_axis_name="c", subcore_axis_name="s",
                                    num_cores=NC, num_subcores=NS),
        scratch_types=[pltpu.VMEM((G_CHUNK,), jnp.int32),
                       pltpu.VMEM((G_CHUNK,), jnp.float32),
                       pltpu.VMEM_SHARED((_N_EDGES,), jnp.float32),
                       pltpu.VMEM_SHARED((_N_EDGES,), jnp.float32)],
    )


_BT = 12544


def _contract_body(tp_ref, kat_ref, kpar_ref, lat_ref, lpar_ref, mpar_ref,
                   out_ref):
    f32 = jnp.float32
    a_sel = (lax.broadcasted_iota(jnp.int32, (_J, _KF), 0) // (_KF * _MF)
             == lax.broadcasted_iota(jnp.int32, (_J, _KF), 1)).astype(f32)
    b_sel = ((lax.broadcasted_iota(jnp.int32, (_J, _KF), 0) // _MF) % _KF
             == lax.broadcasted_iota(jnp.int32, (_J, _KF), 1)).astype(f32)
    e_sel = (lax.broadcasted_iota(jnp.int32, (_MF, _J), 1) % _MF
             == lax.broadcasted_iota(jnp.int32, (_MF, _J), 0)).astype(f32)

    kp = kat_ref[...] * kpar_ref[...]
    lp = lat_ref[...] * lpar_ref[...]
    k_ext = jnp.dot(a_sel, kp, preferred_element_type=f32)
    l_ext = jnp.dot(b_sel, lp, preferred_element_type=f32)
    prod = tp_ref[...] * k_ext * l_ext
    out_ref[...] = (jnp.dot(e_sel, prod, preferred_element_type=f32)
                    * mpar_ref[...])


def _contract(tp_t, kat_t, kpar_t, lat_t, lpar_t, mpar_t):
    grid = (_T + _BT - 1) // _BT
    return pl.pallas_call(
        _contract_body,
        grid=(grid,),
        in_specs=[
            pl.BlockSpec((_J, _BT), lambda i: (0, i)),
            pl.BlockSpec((_KF, _BT), lambda i: (0, i)),
            pl.BlockSpec((_KF, _BT), lambda i: (0, i)),
            pl.BlockSpec((_KF, _BT), lambda i: (0, i)),
            pl.BlockSpec((_KF, _BT), lambda i: (0, i)),
            pl.BlockSpec((_MF, _BT), lambda i: (0, i)),
        ],
        out_specs=pl.BlockSpec((_MF, _BT), lambda i: (0, i)),
        out_shape=jax.ShapeDtypeStruct((_MF, _T), jnp.float32),
        compiler_params=pltpu.CompilerParams(
            dimension_semantics=("arbitrary",)),
    )(tp_t, kat_t, kpar_t, lat_t, lpar_t, mpar_t)


def _scatter_body(vals, sidx, zeros, out, idx_v, val_v, acc):
    c = lax.axis_index("c")
    s = lax.axis_index("s")
    wid = s * NC + c
    base = wid * S_CHUNK
    pltpu.sync_copy(zeros.at[pl.ds(s * ACC_TILE, ACC_TILE)],
                    val_v.at[pl.ds(0, ACC_TILE)])
    pltpu.sync_copy(val_v.at[pl.ds(0, ACC_TILE)],
                    acc.at[pl.ds(s * ACC_TILE, ACC_TILE)])

    @pl.when(wid < NW - 1)
    def _():
        pltpu.sync_copy(sidx.at[pl.ds(base, S_CHUNK)], idx_v)
        pltpu.sync_copy(vals.at[pl.ds(base, S_CHUNK)], val_v)

    @pl.when(wid == NW - 1)
    def _():
        pltpu.sync_copy(sidx.at[pl.ds(base, S_LASTW)],
                        idx_v.at[pl.ds(0, S_LASTW)])
        pltpu.sync_copy(vals.at[pl.ds(base, S_LASTW)],
                        val_v.at[pl.ds(0, S_LASTW)])
        for i in range((S_CHUNK - S_LASTW) // L):
            idx_v[pl.ds(S_LASTW + i * L, L)] = jnp.zeros((L,), jnp.int32)
            val_v[pl.ds(S_LASTW + i * L, L)] = jnp.zeros((L,), jnp.float32)

    plsc.subcore_barrier()
    pltpu.sync_copy(val_v, acc.at[idx_v], add=True)
    plsc.subcore_barrier()
    pltpu.sync_copy(acc.at[pl.ds(s * ACC_TILE, ACC_TILE)],
                    val_v.at[pl.ds(0, ACC_TILE)])
    pltpu.sync_copy(val_v.at[pl.ds(0, ACC_TILE)],
                    out.at[pl.ds(c * ACC_PAD + s * ACC_TILE, ACC_TILE)])


@functools.cache
def _scatter():
    return pl.kernel(
        _scatter_body,
        out_type=jax.ShapeDtypeStruct((NC * ACC_PAD,), jnp.float32),
        mesh=plsc.VectorSubcoreMesh(core_axis_name="c", subcore_axis_name="s",
                                    num_cores=NC, num_subcores=NS),
        scratch_types=[pltpu.VMEM((S_CHUNK,), jnp.int32),
                       pltpu.VMEM((S_CHUNK,), jnp.float32),
                       pltpu.VMEM_SHARED((ACC_PAD,), jnp.float32)],
    )


def _sum_body(p_ref, o_ref):
    o_ref[...] = p_ref[0, :] + p_ref[1, :]


def _sum_partials(partials):
    return pl.pallas_call(
        _sum_body,
        out_shape=jax.ShapeDtypeStruct((ACC_PAD,), jnp.float32),
    )(partials)


def kernel(k_cochain, l_cochain, k_face_idx, k_face_parity, l_face_idx,
           l_face_parity, m_face_idx, m_face_parity, triple_prod):
    kidx = k_face_idx.T.reshape(-1).astype(jnp.int32)
    lidx = l_face_idx.T.reshape(-1).astype(jnp.int32)
    gk, gl = _gather()(k_cochain, l_cochain, kidx, lidx)
    kat_t = gk.reshape(_KF, _T)
    lat_t = gl.reshape(_KF, _T)

    tp_t = jnp.transpose(triple_prod, (1, 2, 3, 0)).reshape(_J, _T)
    mv_t = _contract(tp_t, kat_t, k_face_parity.T, lat_t, l_face_parity.T,
                     m_face_parity.T)

    vals = mv_t.reshape(-1)
    sidx = m_face_idx.T.reshape(-1).astype(jnp.int32)
    zeros = jnp.zeros((ACC_PAD,), jnp.float32)
    partials = _scatter()(vals, sidx, zeros).reshape(NC, ACC_PAD)
    return _sum_partials(partials)[:_N_TRIS]
